# Initial kernel scaffold; baseline (speedup 1.0000x reference)
import jax
import jax.numpy as jnp
from jax.experimental import pallas as pl


def kernel(input, weight):
    raise NotImplementedError("write your pallas kernel here")


# speedup vs baseline: 1.8563x; 1.8563x over previous
import functools

import jax
import jax.numpy as jnp
from jax import lax
from jax.experimental import pallas as pl
from jax.experimental.pallas import tpu as pltpu
from jax.experimental.pallas import tpu_sc as plsc

NUM_EMBEDDINGS = 1000000
EMBEDDING_DIM = 64
BATCH = 16384
HIST = 50

_ROWS_PER_GATHER = 128
_GATHERS_PER_GROUP = 8
_GROUP_ROWS = _ROWS_PER_GATHER * _GATHERS_PER_GROUP


@functools.lru_cache(maxsize=None)
def _build(total_rows: int, dim: int):
    info = plsc.get_sparse_core_info()
    nc, ns = info.num_cores, info.num_subcores
    nw = nc * ns
    assert total_rows % (nw * _GROUP_ROWS) == 0
    rows_per_w = total_rows // nw
    groups_per_w = rows_per_w // _GROUP_ROWS
    idx_rows_per_w = rows_per_w // _ROWS_PER_GATHER

    mesh = plsc.VectorSubcoreMesh(core_axis_name="c", subcore_axis_name="s")

    @functools.partial(
        pl.kernel,
        mesh=mesh,
        out_type=jax.ShapeDtypeStruct((total_rows, dim), jnp.float32),
        scratch_types=[
            pltpu.VMEM((_GATHERS_PER_GROUP, _ROWS_PER_GATHER), jnp.int32),
            pltpu.VMEM((_GROUP_ROWS, dim), jnp.float32),
            pltpu.SemaphoreType.DMA,
        ],
        compiler_params=pltpu.CompilerParams(use_tc_tiling_on_sc=False),
    )
    def gather_kernel(idx_hbm, table_hbm, out_hbm, idx_v, rows_v, sem):
        wid = lax.axis_index("s") * nc + lax.axis_index("c")
        idx_row0 = wid * idx_rows_per_w
        row0 = wid * rows_per_w

        def group_body(g, carry):
            pltpu.sync_copy(
                idx_hbm.at[pl.ds(idx_row0 + g * _GATHERS_PER_GROUP,
                                 _GATHERS_PER_GROUP)],
                idx_v,
            )
            copies = []
            for j in range(_GATHERS_PER_GROUP):
                copies.append(
                    pltpu.async_copy(
                        table_hbm.at[idx_v.at[j]],
                        rows_v.at[pl.ds(j * _ROWS_PER_GATHER, _ROWS_PER_GATHER)],
                        sem,
                    )
                )
            for c in copies:
                c.wait()
            pltpu.sync_copy(
                rows_v,
                out_hbm.at[pl.ds(row0 + g * _GROUP_ROWS, _GROUP_ROWS)],
            )
            return carry

        lax.fori_loop(0, groups_per_w, group_body, 0)

    return gather_kernel


def kernel(input, weight):
    total_rows = input.shape[0] * input.shape[1]
    idx2d = input.reshape(total_rows // _ROWS_PER_GATHER, _ROWS_PER_GATHER)
    out = _build(total_rows, weight.shape[1])(idx2d, weight)
    return out.reshape(input.shape[0], input.shape[1], weight.shape[1])


# speedup vs baseline: 1.8716x; 1.0082x over previous
import functools

import jax
import jax.numpy as jnp
from jax import lax
from jax.experimental import pallas as pl
from jax.experimental.pallas import tpu as pltpu
from jax.experimental.pallas import tpu_sc as plsc

_ROWS_PER_GATHER = 128
_GATHERS_PER_GROUP = 2
_GROUP_ROWS = _ROWS_PER_GATHER * _GATHERS_PER_GROUP
_NBUF = 4
_FIRE_AHEAD = 2


@functools.lru_cache(maxsize=None)
def _build(total_rows: int, dim: int):
    info = plsc.get_sparse_core_info()
    nc, ns = info.num_cores, info.num_subcores
    nw = nc * ns
    assert total_rows % (nw * _GROUP_ROWS * _NBUF) == 0
    rows_per_w = total_rows // nw
    groups_per_w = rows_per_w // _GROUP_ROWS
    idx_rows_per_w = rows_per_w // _ROWS_PER_GATHER
    outer_iters = groups_per_w // _NBUF

    mesh = plsc.VectorSubcoreMesh(core_axis_name="c", subcore_axis_name="s")

    @functools.partial(
        pl.kernel,
        mesh=mesh,
        out_type=jax.ShapeDtypeStruct((total_rows, dim), jnp.float32),
        scratch_types=[
            pltpu.VMEM((idx_rows_per_w, _ROWS_PER_GATHER), jnp.int32),
            pltpu.VMEM((_NBUF, _GROUP_ROWS, dim), jnp.float32),
        ]
        + [pltpu.SemaphoreType.DMA] * (2 * _NBUF),
        compiler_params=pltpu.CompilerParams(use_tc_tiling_on_sc=False),
    )
    def gather_kernel(idx_hbm, table_hbm, out_hbm, idx_v, rows_v, *sems):
        in_sems, out_sems = sems[:_NBUF], sems[_NBUF:]
        wid = lax.axis_index("s") * nc + lax.axis_index("c")
        idx_row0 = wid * idx_rows_per_w
        row0 = wid * rows_per_w

        pltpu.sync_copy(idx_hbm.at[pl.ds(idx_row0, idx_rows_per_w)], idx_v)

        def fire(buf, g):
            for j in range(_GATHERS_PER_GROUP):
                pltpu.async_copy(
                    table_hbm.at[idx_v.at[g * _GATHERS_PER_GROUP + j]],
                    rows_v.at[buf].at[pl.ds(j * _ROWS_PER_GATHER,
                                            _ROWS_PER_GATHER)],
                    in_sems[buf],
                )

        def wait_gathers(buf, g):
            for j in range(_GATHERS_PER_GROUP):
                pltpu.make_async_copy(
                    table_hbm.at[idx_v.at[g * _GATHERS_PER_GROUP + j]],
                    rows_v.at[buf].at[pl.ds(j * _ROWS_PER_GATHER,
                                            _ROWS_PER_GATHER)],
                    in_sems[buf],
                ).wait()

        def start_store(buf, g):
            pltpu.async_copy(
                rows_v.at[buf],
                out_hbm.at[pl.ds(row0 + g * _GROUP_ROWS, _GROUP_ROWS)],
                out_sems[buf],
            )

        def wait_store(buf, g):
            pltpu.make_async_copy(
                rows_v.at[buf],
                out_hbm.at[pl.ds(row0 + g * _GROUP_ROWS, _GROUP_ROWS)],
                out_sems[buf],
            ).wait()

        for g in range(_FIRE_AHEAD):
            fire(g, g)

        def outer_body(p, carry):
            for b in range(_NBUF):
                g = p * _NBUF + b
                bn = (b + _FIRE_AHEAD) % _NBUF
                if b >= _NBUF - _FIRE_AHEAD:
                    wait_store(bn, g + _FIRE_AHEAD - _NBUF)

                    @pl.when(p < outer_iters - 1)
                    def _():
                        fire(bn, g + _FIRE_AHEAD)
                else:

                    @pl.when(p > 0)
                    def _():
                        wait_store(bn, g + _FIRE_AHEAD - _NBUF)

                    fire(bn, g + _FIRE_AHEAD)
                wait_gathers(b, g)
                start_store(b, g)
            return carry

        lax.fori_loop(0, outer_iters, outer_body, 0)

        for g in range(groups_per_w - _FIRE_AHEAD, groups_per_w):
            wait_store(g % _NBUF, g)

    return gather_kernel


def kernel(input, weight):
    total_rows = input.shape[0] * input.shape[1]
    idx2d = input.reshape(total_rows // _ROWS_PER_GATHER, _ROWS_PER_GATHER)
    out = _build(total_rows, weight.shape[1])(idx2d, weight)
    return out.reshape(input.shape[0], input.shape[1], weight.shape[1])
